# baseline (speedup 1.0000x reference)
import jax
import jax.numpy as jnp
from jax import lax
from jax.experimental import pallas as pl
from jax.experimental.pallas import tpu as pltpu
from jax.experimental.pallas import tpu_sc as plsc

B = 16384
EMB = 64
NUM_MOVIES = 1000000
NC = 2
NS = 16
L = 16
NW = NC * NS
BPW = B // NW
NCHUNK = 4
CROWS = BPW // NCHUNK
NGROUPS = BPW // L

MV_BLK = 28672
MV_GRID = -(-NUM_MOVIES // MV_BLK)

UOFF = 0
GOFF = 16
YOFF = 48
PROJ_N = 160

_SC_PARAMS = pltpu.CompilerParams(needs_layout_passes=False,
                                  use_tc_tiling_on_sc=False)
_SC_MESH = dict(core_axis_name="c", subcore_axis_name="s")


def _tc_mv_body(w_all, xt, utT, gtT, ytT, b, out, proj):
    out[...] = jnp.dot(w_all[0:1, 64:128], xt[...],
                       preferred_element_type=jnp.float32)

    @pl.when(pl.program_id(0) == 0)
    def _():
        proj[0:1, UOFF:UOFF + 5] = jnp.dot(
            w_all[0:1, 0:64], utT[...], preferred_element_type=jnp.float32)
        proj[0:1, GOFF:GOFF + 31] = jnp.dot(
            w_all[0:1, 128:192], gtT[...], preferred_element_type=jnp.float32)
        proj[0:1, YOFF:YOFF + 101] = jnp.dot(
            w_all[0:1, 192:256], ytT[...],
            preferred_element_type=jnp.float32) + b[0, 0]


def _sc_body(mflat_hbm, uidx_hbm, gidx_hbm, yidx_hbm, proj_hbm,
             p_hbm, out_hbm, hi_v, mflat_v, uidx_v, gidx_v, yidx_v,
             proj_v, prows_v, out_v, sem):
    wid = lax.axis_index("s") * NC + lax.axis_index("c")
    base = wid * BPW

    pltpu.sync_copy(mflat_hbm.at[pl.ds(base, BPW)], mflat_v)
    pltpu.sync_copy(uidx_hbm.at[pl.ds(base, BPW)], uidx_v)
    pltpu.sync_copy(gidx_hbm.at[pl.ds(base, BPW)], gidx_v)
    pltpu.sync_copy(yidx_hbm.at[pl.ds(base, BPW)], yidx_v)
    pltpu.sync_copy(proj_hbm, proj_v)

    for j in range(NCHUNK):
        for k in range(CROWS // L):
            hi_v[j, pl.ds(k * L, L)] = lax.shift_right_logical(
                mflat_v[pl.ds(j * CROWS + k * L, L)], 4)

    descs = [
        pltpu.async_copy(p_hbm.at[hi_v.at[j]],
                         prows_v.at[pl.ds(j * CROWS, CROWS)], sem)
        for j in range(NCHUNK)
    ]
    for d in descs:
        d.wait()

    riota = lax.iota(jnp.int32, L)

    def group(g, carry):
        row0 = g * L
        rvec = riota + row0
        lovec = mflat_v[pl.ds(row0, L)] & 15
        acc = plsc.load_gather(prows_v, [rvec, lovec])
        uvec = uidx_v[pl.ds(row0, L)] + UOFF
        gvec = gidx_v[pl.ds(row0, L)] + GOFF
        yvec = yidx_v[pl.ds(row0, L)] + YOFF
        acc = acc + plsc.load_gather(proj_v, [uvec])
        acc = acc + plsc.load_gather(proj_v, [gvec])
        acc = acc + plsc.load_gather(proj_v, [yvec])
        out_v[pl.ds(row0, L)] = acc
        return carry

    lax.fori_loop(0, NGROUPS, group, 0)
    pltpu.sync_copy(out_v, out_hbm.at[pl.ds(base, BPW)])


def kernel(user, movie, genre, year, user_table, movie_table, genre_table,
           year_table, fc_w, fc_b):
    user = user.astype(jnp.int32)
    movie = movie.astype(jnp.int32)
    genre = genre.astype(jnp.int32)
    year = year.astype(jnp.int32)

    p, proj2 = pl.pallas_call(
        _tc_mv_body,
        grid=(MV_GRID,),
        in_specs=[
            pl.BlockSpec((1, 256), lambda i: (0, 0)),
            pl.BlockSpec((EMB, MV_BLK), lambda i: (0, i)),
            pl.BlockSpec((EMB, 5), lambda i: (0, 0)),
            pl.BlockSpec((EMB, 31), lambda i: (0, 0)),
            pl.BlockSpec((EMB, 101), lambda i: (0, 0)),
            pl.BlockSpec((1, 1), lambda i: (0, 0)),
        ],
        out_specs=[
            pl.BlockSpec((1, MV_BLK), lambda i: (0, i)),
            pl.BlockSpec((1, PROJ_N), lambda i: (0, 0)),
        ],
        out_shape=[
            jax.ShapeDtypeStruct((1, MV_GRID * MV_BLK), jnp.float32),
            jax.ShapeDtypeStruct((1, PROJ_N), jnp.float32),
        ],
    )(fc_w.reshape(1, 256), movie_table.T, user_table.T, genre_table.T,
      year_table.T, fc_b.reshape(1, 1))
    p16 = p.reshape(MV_GRID * MV_BLK // 16, 16)
    proj = proj2.reshape(PROJ_N)

    sc = pl.kernel(
        _sc_body,
        mesh=plsc.VectorSubcoreMesh(**_SC_MESH),
        compiler_params=_SC_PARAMS,
        out_type=jax.ShapeDtypeStruct((B,), jnp.float32),
        scratch_types=[
            pltpu.VMEM((NCHUNK, CROWS), jnp.int32),
            pltpu.VMEM((BPW,), jnp.int32),
            pltpu.VMEM((BPW,), jnp.int32),
            pltpu.VMEM((BPW,), jnp.int32),
            pltpu.VMEM((BPW,), jnp.int32),
            pltpu.VMEM((PROJ_N,), jnp.float32),
            pltpu.VMEM((BPW, L), jnp.float32),
            pltpu.VMEM((BPW,), jnp.float32),
            pltpu.SemaphoreType.DMA,
        ],
    )
    out = sc(movie, user, genre, year, proj, p16)
    return out.reshape(B, 1)


# speedup vs baseline: 1.0277x; 1.0277x over previous
import jax
import jax.numpy as jnp
from jax import lax
from jax.experimental import pallas as pl
from jax.experimental.pallas import tpu as pltpu
from jax.experimental.pallas import tpu_sc as plsc

B = 16384
EMB = 64
NUM_MOVIES = 1000000
NC = 2
NS = 16
L = 16
NW = NC * NS
BPW = B // NW
NCHUNK = 4
CROWS = BPW // NCHUNK
NGROUPS = BPW // L

MV_BLK = 24576
MV_GRID = -(-NUM_MOVIES // MV_BLK)

UOFF = 0
GOFF = 16
YOFF = 48
PROJ_N = 160

_SC_PARAMS = pltpu.CompilerParams(needs_layout_passes=False,
                                  use_tc_tiling_on_sc=False)
_SC_MESH = dict(core_axis_name="c", subcore_axis_name="s")


def _tc_mv_body(w_all, xt, utT, gtT, ytT, b, out, proj):
    out[...] = jnp.dot(w_all[0:1, 64:128], xt[...],
                       preferred_element_type=jnp.float32)

    @pl.when(pl.program_id(0) == 0)
    def _():
        proj[0:1, UOFF:UOFF + 5] = jnp.dot(
            w_all[0:1, 0:64], utT[...], preferred_element_type=jnp.float32)
        proj[0:1, GOFF:GOFF + 31] = jnp.dot(
            w_all[0:1, 128:192], gtT[...], preferred_element_type=jnp.float32)
        proj[0:1, YOFF:YOFF + 101] = jnp.dot(
            w_all[0:1, 192:256], ytT[...],
            preferred_element_type=jnp.float32) + b[0, 0]


def _sc_body(mflat_hbm, uidx_hbm, gidx_hbm, yidx_hbm, proj_hbm,
             p_hbm, out_hbm, hi_v, mflat_v, uidx_v, gidx_v, yidx_v,
             proj_v, prows_v, out_v, msem, ssem, gsem0, gsem1, gsem2, gsem3):
    wid = lax.axis_index("s") * NC + lax.axis_index("c")
    base = wid * BPW

    mdesc = pltpu.async_copy(mflat_hbm.at[pl.ds(base, BPW)], mflat_v, msem)
    stage = [
        pltpu.async_copy(uidx_hbm.at[pl.ds(base, BPW)], uidx_v, ssem),
        pltpu.async_copy(gidx_hbm.at[pl.ds(base, BPW)], gidx_v, ssem),
        pltpu.async_copy(yidx_hbm.at[pl.ds(base, BPW)], yidx_v, ssem),
        pltpu.async_copy(proj_hbm, proj_v, ssem),
    ]
    mdesc.wait()

    for j in range(NCHUNK):
        for k in range(CROWS // L):
            hi_v[j, pl.ds(k * L, L)] = lax.shift_right_logical(
                mflat_v[pl.ds(j * CROWS + k * L, L)], 4)

    gsems = [gsem0, gsem1, gsem2, gsem3]
    descs = [
        pltpu.async_copy(p_hbm.at[hi_v.at[j]],
                         prows_v.at[pl.ds(j * CROWS, CROWS)], gsems[j])
        for j in range(NCHUNK)
    ]
    for d in stage:
        d.wait()

    riota = lax.iota(jnp.int32, L)

    def group(g, carry):
        row0 = g * L
        rvec = riota + row0
        lovec = mflat_v[pl.ds(row0, L)] & 15
        acc = plsc.load_gather(prows_v, [rvec, lovec])
        uvec = uidx_v[pl.ds(row0, L)] + UOFF
        gvec = gidx_v[pl.ds(row0, L)] + GOFF
        yvec = yidx_v[pl.ds(row0, L)] + YOFF
        acc = acc + plsc.load_gather(proj_v, [uvec])
        acc = acc + plsc.load_gather(proj_v, [gvec])
        acc = acc + plsc.load_gather(proj_v, [yvec])
        out_v[pl.ds(row0, L)] = acc
        return carry

    gpc = NGROUPS // NCHUNK
    for j in range(NCHUNK):
        descs[j].wait()
        lax.fori_loop(j * gpc, (j + 1) * gpc, group, 0)
    pltpu.sync_copy(out_v, out_hbm.at[pl.ds(base, BPW)])


def kernel(user, movie, genre, year, user_table, movie_table, genre_table,
           year_table, fc_w, fc_b):
    user = user.astype(jnp.int32)
    movie = movie.astype(jnp.int32)
    genre = genre.astype(jnp.int32)
    year = year.astype(jnp.int32)

    p, proj2 = pl.pallas_call(
        _tc_mv_body,
        grid=(MV_GRID,),
        in_specs=[
            pl.BlockSpec((1, 256), lambda i: (0, 0)),
            pl.BlockSpec((EMB, MV_BLK), lambda i: (0, i)),
            pl.BlockSpec((EMB, 5), lambda i: (0, 0)),
            pl.BlockSpec((EMB, 31), lambda i: (0, 0)),
            pl.BlockSpec((EMB, 101), lambda i: (0, 0)),
            pl.BlockSpec((1, 1), lambda i: (0, 0)),
        ],
        out_specs=[
            pl.BlockSpec((1, MV_BLK), lambda i: (0, i)),
            pl.BlockSpec((1, PROJ_N), lambda i: (0, 0)),
        ],
        out_shape=[
            jax.ShapeDtypeStruct((1, MV_GRID * MV_BLK), jnp.float32),
            jax.ShapeDtypeStruct((1, PROJ_N), jnp.float32),
        ],
    )(fc_w.reshape(1, 256), movie_table.T, user_table.T, genre_table.T,
      year_table.T, fc_b.reshape(1, 1))
    p16 = p.reshape(MV_GRID * MV_BLK // 16, 16)
    proj = proj2.reshape(PROJ_N)

    sc = pl.kernel(
        _sc_body,
        mesh=plsc.VectorSubcoreMesh(**_SC_MESH),
        compiler_params=_SC_PARAMS,
        out_type=jax.ShapeDtypeStruct((B,), jnp.float32),
        scratch_types=[
            pltpu.VMEM((NCHUNK, CROWS), jnp.int32),
            pltpu.VMEM((BPW,), jnp.int32),
            pltpu.VMEM((BPW,), jnp.int32),
            pltpu.VMEM((BPW,), jnp.int32),
            pltpu.VMEM((BPW,), jnp.int32),
            pltpu.VMEM((PROJ_N,), jnp.float32),
            pltpu.VMEM((BPW, L), jnp.float32),
            pltpu.VMEM((BPW,), jnp.float32),
            pltpu.SemaphoreType.DMA,
            pltpu.SemaphoreType.DMA,
            pltpu.SemaphoreType.DMA,
            pltpu.SemaphoreType.DMA,
            pltpu.SemaphoreType.DMA,
            pltpu.SemaphoreType.DMA,
        ],
    )
    out = sc(movie, user, genre, year, proj, p16)
    return out.reshape(B, 1)


# speedup vs baseline: 1.0328x; 1.0050x over previous
import jax
import jax.numpy as jnp
from jax import lax
from jax.experimental import pallas as pl
from jax.experimental.pallas import tpu as pltpu
from jax.experimental.pallas import tpu_sc as plsc

B = 16384
EMB = 64
NUM_MOVIES = 1000000
NC = 2
NS = 16
L = 16
NW = NC * NS
BPW = B // NW
NCHUNK = 4
CROWS = BPW // NCHUNK
NGROUPS = BPW // L

MV_BLK = 26624
MV_GRID = -(-NUM_MOVIES // MV_BLK)

UOFF = 0
GOFF = 16
YOFF = 48
PROJ_N = 160

_SC_PARAMS = pltpu.CompilerParams(needs_layout_passes=False,
                                  use_tc_tiling_on_sc=False)
_SC_MESH = dict(core_axis_name="c", subcore_axis_name="s")


def _tc_mv_body(w_all, xt, smT, b, out, proj):
    out[...] = jnp.dot(w_all[0:1, 64:128], xt[...],
                       preferred_element_type=jnp.float32)

    @pl.when(pl.program_id(0) == 0)
    def _():
        proj[0:1, UOFF:UOFF + 5] = jnp.dot(
            w_all[0:1, 0:64], smT[:, 0:5],
            preferred_element_type=jnp.float32)
        proj[0:1, GOFF:GOFF + 31] = jnp.dot(
            w_all[0:1, 128:192], smT[:, 5:36],
            preferred_element_type=jnp.float32)
        proj[0:1, YOFF:YOFF + 101] = jnp.dot(
            w_all[0:1, 192:256], smT[:, 36:137],
            preferred_element_type=jnp.float32) + b[0, 0]


def _sc_body(mflat_hbm, uidx_hbm, gidx_hbm, yidx_hbm, proj_hbm,
             p_hbm, out_hbm, hi_v, mflat_v, uidx_v, gidx_v, yidx_v,
             proj_v, prows_v, out_v, msem, ssem, gsem0, gsem1, gsem2, gsem3):
    wid = lax.axis_index("s") * NC + lax.axis_index("c")
    base = wid * BPW

    mdesc = pltpu.async_copy(mflat_hbm.at[pl.ds(base, BPW)], mflat_v, msem)
    stage = [
        pltpu.async_copy(uidx_hbm.at[pl.ds(base, BPW)], uidx_v, ssem),
        pltpu.async_copy(gidx_hbm.at[pl.ds(base, BPW)], gidx_v, ssem),
        pltpu.async_copy(yidx_hbm.at[pl.ds(base, BPW)], yidx_v, ssem),
        pltpu.async_copy(proj_hbm, proj_v, ssem),
    ]
    mdesc.wait()

    for j in range(NCHUNK):
        for k in range(CROWS // L):
            hi_v[j, pl.ds(k * L, L)] = lax.shift_right_logical(
                mflat_v[pl.ds(j * CROWS + k * L, L)], 4)

    gsems = [gsem0, gsem1, gsem2, gsem3]
    descs = [
        pltpu.async_copy(p_hbm.at[hi_v.at[j]],
                         prows_v.at[pl.ds(j * CROWS, CROWS)], gsems[j])
        for j in range(NCHUNK)
    ]
    for d in stage:
        d.wait()

    riota = lax.iota(jnp.int32, L)

    def group(g, carry):
        row0 = g * L
        rvec = riota + row0
        lovec = mflat_v[pl.ds(row0, L)] & 15
        acc = plsc.load_gather(prows_v, [rvec, lovec])
        uvec = uidx_v[pl.ds(row0, L)] + UOFF
        gvec = gidx_v[pl.ds(row0, L)] + GOFF
        yvec = yidx_v[pl.ds(row0, L)] + YOFF
        acc = acc + plsc.load_gather(proj_v, [uvec])
        acc = acc + plsc.load_gather(proj_v, [gvec])
        acc = acc + plsc.load_gather(proj_v, [yvec])
        out_v[pl.ds(row0, L)] = acc
        return carry

    gpc = NGROUPS // NCHUNK
    for j in range(NCHUNK):
        descs[j].wait()
        lax.fori_loop(j * gpc, (j + 1) * gpc, group, 0)
    pltpu.sync_copy(out_v, out_hbm.at[pl.ds(base, BPW)])


def kernel(user, movie, genre, year, user_table, movie_table, genre_table,
           year_table, fc_w, fc_b):
    user = user.astype(jnp.int32)
    movie = movie.astype(jnp.int32)
    genre = genre.astype(jnp.int32)
    year = year.astype(jnp.int32)

    p, proj2 = pl.pallas_call(
        _tc_mv_body,
        grid=(MV_GRID,),
        in_specs=[
            pl.BlockSpec((1, 256), lambda i: (0, 0)),
            pl.BlockSpec((EMB, MV_BLK), lambda i: (0, i)),
            pl.BlockSpec((EMB, 137), lambda i: (0, 0)),
            pl.BlockSpec((1, 1), lambda i: (0, 0)),
        ],
        out_specs=[
            pl.BlockSpec((1, MV_BLK), lambda i: (0, i)),
            pl.BlockSpec((1, PROJ_N), lambda i: (0, 0)),
        ],
        out_shape=[
            jax.ShapeDtypeStruct((1, MV_GRID * MV_BLK), jnp.float32),
            jax.ShapeDtypeStruct((1, PROJ_N), jnp.float32),
        ],
    )(fc_w.reshape(1, 256), movie_table.T,
      jnp.concatenate([user_table.T, genre_table.T, year_table.T], axis=1),
      fc_b.reshape(1, 1))
    p16 = p.reshape(MV_GRID * MV_BLK // 16, 16)
    proj = proj2.reshape(PROJ_N)

    sc = pl.kernel(
        _sc_body,
        mesh=plsc.VectorSubcoreMesh(**_SC_MESH),
        compiler_params=_SC_PARAMS,
        out_type=jax.ShapeDtypeStruct((B,), jnp.float32),
        scratch_types=[
            pltpu.VMEM((NCHUNK, CROWS), jnp.int32),
            pltpu.VMEM((BPW,), jnp.int32),
            pltpu.VMEM((BPW,), jnp.int32),
            pltpu.VMEM((BPW,), jnp.int32),
            pltpu.VMEM((BPW,), jnp.int32),
            pltpu.VMEM((PROJ_N,), jnp.float32),
            pltpu.VMEM((BPW, L), jnp.float32),
            pltpu.VMEM((BPW,), jnp.float32),
            pltpu.SemaphoreType.DMA,
            pltpu.SemaphoreType.DMA,
            pltpu.SemaphoreType.DMA,
            pltpu.SemaphoreType.DMA,
            pltpu.SemaphoreType.DMA,
            pltpu.SemaphoreType.DMA,
        ],
    )
    out = sc(movie, user, genre, year, proj, p16)
    return out.reshape(B, 1)
